# Initial kernel scaffold; baseline (speedup 1.0000x reference)
import jax
import jax.numpy as jnp
from jax.experimental import pallas as pl


def kernel(x):
    raise NotImplementedError("write your pallas kernel here")


# speedup vs baseline: 2.9512x; 2.9512x over previous
import jax
import jax.numpy as jnp
from jax.experimental import pallas as pl
from jax.experimental.pallas import tpu as pltpu

_SLICES_PER_BLOCK = 8


def _fuzzy_pool_body(x_ref, o_ref):
    x = x_ref[...]
    r, h, w = x.shape
    fz = x * jnp.exp(x * x * -0.5)
    s4 = fz.reshape(r, h // 2, 2, w)
    s = s4[:, :, 0, :] + s4[:, :, 1, :]
    rows = jax.lax.broadcasted_iota(jnp.int32, (w, w // 2), 0)
    cols = jax.lax.broadcasted_iota(jnp.int32, (w, w // 2), 1)
    p = jnp.where(rows // 2 == cols, 0.25, 0.0).astype(x.dtype)
    o_ref[...] = jax.lax.dot_general(
        s, p, (((2,), (0,)), ((), ())),
        preferred_element_type=jnp.float32)


def kernel(x):
    b, c, h, w = x.shape
    oh, ow = h // 2, w // 2
    n = b * c
    xf = x.reshape(n, h, w)
    r = _SLICES_PER_BLOCK
    out = pl.pallas_call(
        _fuzzy_pool_body,
        grid=(n // r,),
        in_specs=[pl.BlockSpec((r, h, w), lambda i: (i, 0, 0))],
        out_specs=pl.BlockSpec((r, oh, ow), lambda i: (i, 0, 0)),
        out_shape=jax.ShapeDtypeStruct((n, oh, ow), x.dtype),
        compiler_params=pltpu.CompilerParams(
            dimension_semantics=("parallel",),
        ),
    )(xf)
    return out.reshape(b, c, oh, ow)


# speedup vs baseline: 3.7927x; 1.2851x over previous
import jax
import jax.numpy as jnp
from jax.experimental import pallas as pl
from jax.experimental.pallas import tpu as pltpu

_SLICES_PER_BLOCK = 32


def _fuzzy_pool_body(x_ref, o_ref):
    x = x_ref[...]
    r, h, w = x.shape
    fz = x * jnp.exp(x * x * -0.5)
    s4 = fz.reshape(r, h // 2, 2, w)
    s = s4[:, :, 0, :] + s4[:, :, 1, :]
    rows = jax.lax.broadcasted_iota(jnp.int32, (w, w // 2), 0)
    cols = jax.lax.broadcasted_iota(jnp.int32, (w, w // 2), 1)
    p = jnp.where(rows // 2 == cols, 0.25, 0.0).astype(x.dtype)
    o_ref[...] = jax.lax.dot_general(
        s, p, (((2,), (0,)), ((), ())),
        preferred_element_type=jnp.float32)


def kernel(x):
    b, c, h, w = x.shape
    oh, ow = h // 2, w // 2
    n = b * c
    xf = x.reshape(n, h, w)
    r = _SLICES_PER_BLOCK
    out = pl.pallas_call(
        _fuzzy_pool_body,
        grid=(n // r,),
        in_specs=[pl.BlockSpec((r, h, w), lambda i: (i, 0, 0))],
        out_specs=pl.BlockSpec((r, oh, ow), lambda i: (i, 0, 0)),
        out_shape=jax.ShapeDtypeStruct((n, oh, ow), x.dtype),
        compiler_params=pltpu.CompilerParams(
            dimension_semantics=("parallel",),
        ),
    )(xf)
    return out.reshape(b, c, oh, ow)


# speedup vs baseline: 3.8164x; 1.0062x over previous
import jax
import jax.numpy as jnp
from jax.experimental import pallas as pl
from jax.experimental.pallas import tpu as pltpu

_SLICES_PER_BLOCK = 64


def _fuzzy_pool_body(x_ref, o_ref):
    x = x_ref[...]
    r, h, w = x.shape
    fz = x * jnp.exp(x * x * -0.5)
    s4 = fz.reshape(r, h // 2, 2, w)
    s = s4[:, :, 0, :] + s4[:, :, 1, :]
    rows = jax.lax.broadcasted_iota(jnp.int32, (w, w // 2), 0)
    cols = jax.lax.broadcasted_iota(jnp.int32, (w, w // 2), 1)
    p = jnp.where(rows // 2 == cols, 0.25, 0.0).astype(x.dtype)
    o_ref[...] = jax.lax.dot_general(
        s, p, (((2,), (0,)), ((), ())),
        preferred_element_type=jnp.float32)


def kernel(x):
    b, c, h, w = x.shape
    oh, ow = h // 2, w // 2
    n = b * c
    xf = x.reshape(n, h, w)
    r = _SLICES_PER_BLOCK
    out = pl.pallas_call(
        _fuzzy_pool_body,
        grid=(n // r,),
        in_specs=[pl.BlockSpec((r, h, w), lambda i: (i, 0, 0))],
        out_specs=pl.BlockSpec((r, oh, ow), lambda i: (i, 0, 0)),
        out_shape=jax.ShapeDtypeStruct((n, oh, ow), x.dtype),
        compiler_params=pltpu.CompilerParams(
            dimension_semantics=("parallel",),
        ),
    )(xf)
    return out.reshape(b, c, oh, ow)
